# Initial kernel scaffold; baseline (speedup 1.0000x reference)
import jax
import jax.numpy as jnp
from jax.experimental import pallas as pl


def kernel(input_ids, weights):
    raise NotImplementedError("write your pallas kernel here")


# speedup vs baseline: 1.7650x; 1.7650x over previous
import functools

import jax
import jax.numpy as jnp
from jax import lax
from jax.experimental import pallas as pl
from jax.experimental.pallas import tpu as pltpu
from jax.experimental.pallas import tpu_sc as plsc

_PAD = 1

_NC = 2
_NS = 16
_NW = _NC * _NS


def _positions_body(ids_ref, out_ref):
    ids = ids_ref[...]
    mask = (ids != _PAD).astype(jnp.int32)
    bsz, seq_len = ids.shape
    csum = mask
    k = 1
    while k < seq_len:
        shifted = jnp.concatenate(
            [jnp.zeros((bsz, k), jnp.int32), csum[:, :-k]], axis=1
        )
        csum = csum + shifted
        k *= 2
    out_ref[...] = csum * mask + _PAD


def _compute_position_ids(input_ids):
    return pl.pallas_call(
        _positions_body,
        out_shape=jax.ShapeDtypeStruct(input_ids.shape, jnp.int32),
    )(input_ids)


def _make_sc_gather(n_rows, d, n_chunks, chunk):
    b_per_w = n_chunks * chunk
    assert b_per_w * _NW == n_rows
    mesh = plsc.VectorSubcoreMesh(core_axis_name="c", subcore_axis_name="s")

    @functools.partial(
        pl.kernel,
        mesh=mesh,
        out_type=jax.ShapeDtypeStruct((n_rows, d), jnp.float32),
        scratch_types=[
            pltpu.VMEM((n_chunks, chunk), jnp.int32),
            pltpu.VMEM((chunk, d), jnp.float32),
            pltpu.SemaphoreType.DMA,
        ],
    )
    def sc_gather(table_hbm, idx_hbm, out_hbm, idx_v, rows_v, sem):
        wid = lax.axis_index("s") * _NC + lax.axis_index("c")
        base = wid * b_per_w
        pltpu.sync_copy(idx_hbm.at[wid], idx_v)
        for c in range(n_chunks):
            pltpu.async_copy(table_hbm.at[idx_v.at[c]], rows_v, sem).wait()
            pltpu.sync_copy(rows_v, out_hbm.at[pl.ds(base + c * chunk, chunk)])

    return sc_gather


def kernel(input_ids, weights):
    bsz, seq_len = input_ids.shape
    n_rows = bsz * seq_len
    d = weights.shape[1]

    position_ids = _compute_position_ids(input_ids)

    n_chunks, chunk = 8, 64
    idx = position_ids.reshape(_NW, n_chunks, chunk)
    out = _make_sc_gather(n_rows, d, n_chunks, chunk)(weights, idx)
    return out.reshape(bsz, seq_len, d)
